# baseline (speedup 1.0000x reference)
import functools

import jax
import jax.numpy as jnp
from jax import lax
from jax.experimental import pallas as pl
from jax.experimental.pallas import tpu as pltpu
from jax.experimental.pallas import tpu_sc as plsc


_INFO = plsc.get_sparse_core_info()
_NC, _NS = _INFO.num_cores, _INFO.num_subcores
_NW = _NC * _NS


def _make_sc_copy(rows, dim, dtype):
    rows_per_w = rows // _NW
    mesh = plsc.VectorSubcoreMesh(core_axis_name="c", subcore_axis_name="s")

    @functools.partial(
        pl.kernel,
        mesh=mesh,
        out_type=jax.ShapeDtypeStruct((rows, dim), dtype),
        scratch_types=[pltpu.SemaphoreType.DMA],
    )
    def sc_copy(table_hbm, out_hbm, sem):
        wid = lax.axis_index("s") * _NC + lax.axis_index("c")
        base = wid * rows_per_w
        copy = pltpu.make_async_copy(
            table_hbm.at[pl.ds(base, rows_per_w)],
            out_hbm.at[pl.ds(base, rows_per_w)],
            sem,
        )
        copy.start()
        copy.wait()

    return sc_copy


def kernel(x, emb_weight):
    rows, dim = emb_weight.shape
    assert x.shape[1] == rows and rows % _NW == 0
    return _make_sc_copy(rows, dim, emb_weight.dtype)(emb_weight)


# speedup vs baseline: 23.3051x; 23.3051x over previous
import functools

import jax
import jax.numpy as jnp
from jax import lax
from jax.experimental import pallas as pl
from jax.experimental.pallas import tpu as pltpu
from jax.experimental.pallas import tpu_sc as plsc


_INFO = plsc.get_sparse_core_info()
_NC, _NS = _INFO.num_cores, _INFO.num_subcores
_NW = _NC * _NS

_CHUNK_ROWS = 32


def _make_sc_copy(rows, dim, dtype):
    rows_per_w = rows // _NW
    n_chunks = rows_per_w // _CHUNK_ROWS
    mesh = plsc.VectorSubcoreMesh(core_axis_name="c", subcore_axis_name="s")

    @functools.partial(
        pl.kernel,
        mesh=mesh,
        out_type=jax.ShapeDtypeStruct((rows, dim), dtype),
        scratch_types=[
            pltpu.VMEM((2, _CHUNK_ROWS, dim), dtype),
            pltpu.SemaphoreType.DMA,
            pltpu.SemaphoreType.DMA,
        ],
    )
    def sc_copy(table_hbm, out_hbm, buf, load_sem, store_sem):
        wid = lax.axis_index("s") * _NC + lax.axis_index("c")
        base = wid * rows_per_w

        def load(g, b):
            return pltpu.make_async_copy(
                table_hbm.at[pl.ds(base + g * _CHUNK_ROWS, _CHUNK_ROWS)],
                buf.at[b],
                load_sem,
            )

        def store(g, b):
            return pltpu.make_async_copy(
                buf.at[b],
                out_hbm.at[pl.ds(base + g * _CHUNK_ROWS, _CHUNK_ROWS)],
                store_sem,
            )

        load(0, 0).start()
        for g in range(n_chunks):
            b = g % 2
            load(g, b).wait()
            store(g, b).start()
            if g + 1 < n_chunks:
                if g >= 1:
                    store(g - 1, 1 - b).wait()
                load(g + 1, 1 - b).start()
        store(n_chunks - 2, n_chunks % 2).wait()
        store(n_chunks - 1, (n_chunks - 1) % 2).wait()

    return sc_copy


def kernel(x, emb_weight):
    rows, dim = emb_weight.shape
    assert x.shape[1] == rows and rows % (_NW * _CHUNK_ROWS) == 0
    return _make_sc_copy(rows, dim, emb_weight.dtype)(emb_weight)
